# baseline (speedup 1.0000x reference)
import functools

import jax
import jax.numpy as jnp
from jax import lax
from jax.experimental import pallas as pl
from jax.experimental.pallas import tpu as pltpu
from jax.experimental.pallas import tpu_sc as plsc

_EMBED = 32
_CHUNK = 128


@functools.lru_cache(maxsize=None)
def _make_gather(n_rows_total):
    info = plsc.get_sparse_core_info()
    nc, ns = info.num_cores, info.num_subcores
    nw = nc * ns
    assert n_rows_total % (nw * 8) == 0
    b_per_w = n_rows_total // nw
    n_chunks = b_per_w // _CHUNK
    assert n_chunks * _CHUNK == b_per_w

    mesh = plsc.VectorSubcoreMesh(core_axis_name="c", subcore_axis_name="s")

    @functools.partial(
        pl.kernel,
        mesh=mesh,
        compiler_params=pltpu.CompilerParams(use_tc_tiling_on_sc=False),
        out_type=jax.ShapeDtypeStruct((n_rows_total, _EMBED), jnp.bfloat16),
        scratch_types=[
            pltpu.VMEM((b_per_w,), jnp.int32),
            pltpu.VMEM((b_per_w, _EMBED), jnp.bfloat16),
            pltpu.SemaphoreType.DMA,
        ],
    )
    def gather_k(tbl_hbm, idx_hbm, out_hbm, idx_v, rows_v, sem):
        wid = lax.axis_index("s") * nc + lax.axis_index("c")
        base = wid * b_per_w
        pltpu.sync_copy(idx_hbm.at[pl.ds(base, b_per_w)], idx_v)

        def body(j, carry):
            sl = pl.ds(j * _CHUNK, _CHUNK)
            pltpu.async_copy(tbl_hbm.at[idx_v.at[sl]], rows_v.at[sl, :], sem).wait()
            return carry

        lax.fori_loop(0, n_chunks, body, 0)
        pltpu.sync_copy(rows_v, out_hbm.at[pl.ds(base, b_per_w)])

    return gather_k


def _mlp_body(*refs):
    r_ref = refs[0]
    out_ref = refs[-1]
    w = refs[1:-1]
    r = r_ref[...].astype(jnp.float32)
    n_units = (len(w) - 2) // 4
    for u in range(n_units):
        w1, b1, w2, b2 = w[4 * u : 4 * u + 4]
        h = jnp.dot(r, w1[...], preferred_element_type=jnp.float32) + b1[...]
        h = jnp.maximum(h, 0.0)
        h = jnp.dot(h, w2[...], preferred_element_type=jnp.float32) + b2[...]
        r = jnp.maximum(r + h, 0.0)
    wd, bd = w[-2], w[-1]
    logit = jnp.dot(r, wd[...], preferred_element_type=jnp.float32) + bd[...]
    out_ref[...] = jax.nn.sigmoid(logit)


def _mlp(r, flat_w, block_b=512):
    batch, stack = r.shape
    grid = (batch // block_b,)
    full = lambda a: pl.BlockSpec(a.shape, lambda i: (0,) * a.ndim)
    in_specs = [pl.BlockSpec((block_b, stack), lambda i: (i, 0))]
    in_specs += [full(a) for a in flat_w]
    return pl.pallas_call(
        _mlp_body,
        grid=grid,
        in_specs=in_specs,
        out_specs=pl.BlockSpec((block_b, 1), lambda i: (i, 0)),
        out_shape=jax.ShapeDtypeStruct((batch, 1), jnp.float32),
    )(r, *flat_w)


def kernel(sparse_inputs, params):
    tables = params["tables"]
    n_fields, vocab, embed = tables.shape
    batch = sparse_inputs.shape[0]
    flat_tbl = tables.reshape(n_fields * vocab, embed).astype(jnp.bfloat16)
    offs = (jnp.arange(n_fields, dtype=jnp.int32) * vocab)[None, :]
    flat_idx = (sparse_inputs.astype(jnp.int32) + offs).reshape(-1)

    rows = _make_gather(batch * n_fields)(flat_tbl, flat_idx)
    r = rows.reshape(batch, n_fields * embed)

    flat_w = []
    for (w1, b1, w2, b2) in params["res"]:
        flat_w += [w1, b1[None, :], w2, b2[None, :]]
    flat_w += [params["Wd"], params["bd"][None, :]]
    return _mlp(r, tuple(flat_w))


# speedup vs baseline: 3.9744x; 3.9744x over previous
import functools

import jax
import jax.numpy as jnp
from jax import lax
from jax.experimental import pallas as pl
from jax.experimental.pallas import tpu as pltpu
from jax.experimental.pallas import tpu_sc as plsc

_LANE = 16
_CV = 5120


@functools.lru_cache(maxsize=None)
def _make_gather(n_fields, embed, vocab, batch):
    info = plsc.get_sparse_core_info()
    nc, ns = info.num_cores, info.num_subcores
    nw = nc * ns
    egroups = embed // 8
    n_units = n_fields * egroups
    units_per_w = -(-n_units // nw)
    v_aligned = (vocab // 128) * 128
    n_full = v_aligned // _CV
    tail_v = v_aligned - n_full * _CV
    n_chunks = n_full + (1 if tail_v else 0)

    mesh = plsc.VectorSubcoreMesh(core_axis_name="c", subcore_axis_name="s")

    @functools.partial(
        pl.kernel,
        mesh=mesh,
        compiler_params=pltpu.CompilerParams(
            use_tc_tiling_on_sc=True, needs_layout_passes=False),
        out_type=jax.ShapeDtypeStruct((n_fields * embed, batch), jnp.float32),
        scratch_types=[
            pltpu.VMEM((batch,), jnp.int32),
            pltpu.VMEM((batch,), jnp.int32),
            pltpu.VMEM((32,), jnp.int32),
            pltpu.VMEM((8, _CV), jnp.float32),
            pltpu.VMEM((8, _CV), jnp.float32),
            pltpu.VMEM((8, 128), jnp.float32),
            pltpu.VMEM((8, batch), jnp.float32),
            pltpu.SemaphoreType.DMA,
            pltpu.SemaphoreType.DMA,
        ],
    )
    def gather_k(tbl_hbm, reg_hbm, sv_hbm, pos_hbm, bnd_hbm, out_hbm,
                 sv_v, pos_v, bnd_v, buf0, buf1, reg_v, slab_v, sem0, sem1):
        wid = lax.axis_index("s") * nc + lax.axis_index("c")
        bufs = (buf0, buf1)
        sems = (sem0, sem1)
        iota16 = lax.iota(jnp.int32, _LANE)

        def chunk_src(f, g8, c):
            v0 = c * _CV
            ext = _CV if c < n_full else tail_v
            return tbl_hbm.at[f, pl.ds(g8, 8), pl.ds(v0, ext)], ext

        def fire(f, g8, c):
            src, ext = chunk_src(f, g8, c)
            pltpu.async_copy(src, bufs[c % 2].at[:, pl.ds(0, ext)], sems[c % 2])

        def wait(f, g8, c):
            src, ext = chunk_src(f, g8, c)
            pltpu.make_async_copy(
                src, bufs[c % 2].at[:, pl.ds(0, ext)], sems[c % 2]).wait()

        def scalar_at(vec_ref, j):
            grp = vec_ref[pl.ds((j // _LANE) * _LANE, _LANE)]
            return jnp.sum(jnp.where(iota16 == (j % _LANE), grp, 0))

        def unit_body(j, carry):
            u = wid + j * nw

            @pl.when(u < n_units)
            def _():
                f = u // egroups
                g8 = pl.multiple_of((u % egroups) * 8, 8)
                pltpu.sync_copy(sv_hbm.at[f], sv_v)
                pltpu.sync_copy(pos_hbm.at[f], pos_v)
                pltpu.sync_copy(bnd_hbm.at[f], bnd_v)
                pltpu.sync_copy(reg_hbm.at[f, pl.ds(g8, 8), :], reg_v)

                def process(buf, v0, v1, lo, hi):
                    def group_body(k, c2):
                        off = pl.multiple_of(k * _LANE, 8)
                        sv16 = sv_v[pl.ds(off, _LANE)]
                        pos16 = pos_v[pl.ds(off, _LANE)]
                        m = jnp.logical_and(sv16 >= v0, sv16 < v1)
                        vloc = jnp.where(m, sv16 - v0, 0)
                        for s in range(8):
                            s16 = jnp.full((_LANE,), s, jnp.int32)
                            vals = plsc.load_gather(buf, [s16, vloc])
                            plsc.store_scatter(slab_v, [s16, pos16], vals, mask=m)
                        return c2

                    lax.fori_loop(lo >> 4, (hi + _LANE - 1) >> 4, group_body, 0)

                fire(f, g8, 0)
                for c in range(n_chunks):
                    if c + 1 < n_chunks:
                        fire(f, g8, c + 1)
                    wait(f, g8, c)
                    v0 = c * _CV
                    v1 = v0 + (_CV if c < n_full else tail_v)
                    process(bufs[c % 2], v0, v1,
                            scalar_at(bnd_v, c), scalar_at(bnd_v, c + 1))
                process(reg_v, v_aligned, vocab,
                        scalar_at(bnd_v, n_chunks), scalar_at(bnd_v, n_chunks + 1))
                row0 = f * embed + g8
                pltpu.sync_copy(slab_v, out_hbm.at[pl.ds(row0, 8), :])

            return carry

        lax.fori_loop(0, units_per_w, unit_body, 0)

    return gather_k


def _mlp_t_body(*refs):
    rt_ref = refs[0]
    out_ref = refs[-1]
    w = refs[1:-1]
    rt = rt_ref[...]
    cdims = (((0,), (0,)), ((), ()))
    n_units = (len(w) - 2) // 4
    for u in range(n_units):
        w1, b1, w2, b2 = w[4 * u : 4 * u + 4]
        h = lax.dot_general(w1[...], rt, cdims, preferred_element_type=jnp.float32)
        h = jnp.maximum(h + b1[...], 0.0)
        h = lax.dot_general(w2[...], h, cdims, preferred_element_type=jnp.float32)
        rt = jnp.maximum(rt + h + b2[...], 0.0)
    wd, bd = w[-2], w[-1]
    logit = lax.dot_general(wd[...], rt, cdims, preferred_element_type=jnp.float32)
    out_ref[...] = jax.nn.sigmoid(logit + bd[...])


def _mlp_t(rt, flat_w, block_c=512):
    stack, batch = rt.shape
    grid = (batch // block_c,)
    full = lambda a: pl.BlockSpec(a.shape, lambda i: (0,) * a.ndim)
    in_specs = [pl.BlockSpec((stack, block_c), lambda i: (0, i))]
    in_specs += [full(a) for a in flat_w]
    return pl.pallas_call(
        _mlp_t_body,
        grid=grid,
        in_specs=in_specs,
        out_specs=pl.BlockSpec((1, block_c), lambda i: (0, i)),
        out_shape=jax.ShapeDtypeStruct((1, batch), jnp.float32),
    )(rt, *flat_w)


def kernel(sparse_inputs, params):
    tables = params["tables"]
    n_fields, vocab, embed = tables.shape
    batch = sparse_inputs.shape[0]
    tbl_t = jnp.transpose(tables, (0, 2, 1))

    idx_t = sparse_inputs.astype(jnp.int32).T
    iota_b = jnp.broadcast_to(
        jnp.arange(batch, dtype=jnp.int32)[None, :], idx_t.shape)
    sv, pos = lax.sort((idx_t, iota_b), dimension=1, num_keys=1)
    v_aligned = (vocab // 128) * 128
    grid = jnp.array(
        list(range(0, v_aligned + 1, _CV))[: v_aligned // _CV + 1]
        + [v_aligned, vocab], dtype=jnp.int32)
    bnd = jax.vmap(lambda row: jnp.searchsorted(row, grid).astype(jnp.int32))(sv)
    bnd = jnp.pad(bnd, ((0, 0), (0, 32 - bnd.shape[1])), mode="edge")
    reg = jnp.pad(tbl_t[:, :, v_aligned:], ((0, 0), (0, 0),
                                            (0, 128 - (vocab - v_aligned))))

    rt = _make_gather(n_fields, embed, vocab, batch)(tbl_t, reg, sv, pos, bnd)

    flat_w = []
    for (w1, b1, w2, b2) in params["res"]:
        flat_w += [w1, b1[:, None], w2, b2[:, None]]
    flat_w += [params["Wd"], params["bd"][:, None]]
    out_t = _mlp_t(rt, tuple(flat_w))
    return out_t.reshape(batch, 1)


# speedup vs baseline: 5.1494x; 1.2956x over previous
import functools

import jax
import jax.numpy as jnp
from jax import lax
from jax.experimental import pallas as pl
from jax.experimental.pallas import tpu as pltpu
from jax.experimental.pallas import tpu_sc as plsc

_LANE = 16
_CV = 5120


@functools.lru_cache(maxsize=None)
def _make_gather(n_fields, embed, vocab, batch):
    info = plsc.get_sparse_core_info()
    nc, ns = info.num_cores, info.num_subcores
    nw = nc * ns
    egroups = embed // 8
    n_units = n_fields * egroups
    units_per_w = -(-n_units // nw)
    v_aligned = (vocab // 128) * 128
    n_full = v_aligned // _CV
    tail_v = v_aligned - n_full * _CV
    n_chunks = n_full + (1 if tail_v else 0)
    pos_bits = batch.bit_length() - 1
    assert (1 << pos_bits) == batch and vocab * batch < 2**31

    mesh = plsc.VectorSubcoreMesh(core_axis_name="c", subcore_axis_name="s")

    @functools.partial(
        pl.kernel,
        mesh=mesh,
        compiler_params=pltpu.CompilerParams(
            use_tc_tiling_on_sc=True, needs_layout_passes=False),
        out_type=jax.ShapeDtypeStruct((n_fields * embed, batch), jnp.float32),
        scratch_types=[
            pltpu.VMEM((batch,), jnp.int32),
            pltpu.VMEM((32,), jnp.int32),
            pltpu.VMEM((8, _CV), jnp.float32),
            pltpu.VMEM((8, _CV), jnp.float32),
            pltpu.VMEM((8, 128), jnp.float32),
            pltpu.VMEM((8, batch), jnp.float32),
            pltpu.SemaphoreType.DMA,
            pltpu.SemaphoreType.DMA,
        ],
    )
    def gather_k(tbl_hbm, reg_hbm, sv_hbm, bnd_hbm, out_hbm,
                 sv_v, bnd_v, buf0, buf1, reg_v, slab_v, sem0, sem1):
        wid = lax.axis_index("s") * nc + lax.axis_index("c")
        bufs = (buf0, buf1)
        sems = (sem0, sem1)
        iota16 = lax.iota(jnp.int32, _LANE)

        def chunk_src(f, g8, c):
            v0 = c * _CV
            ext = _CV if c < n_full else tail_v
            return tbl_hbm.at[f, pl.ds(g8, 8), pl.ds(v0, ext)], ext

        def fire(f, g8, c):
            src, ext = chunk_src(f, g8, c)
            pltpu.async_copy(src, bufs[c % 2].at[:, pl.ds(0, ext)], sems[c % 2])

        def wait(f, g8, c):
            src, ext = chunk_src(f, g8, c)
            pltpu.make_async_copy(
                src, bufs[c % 2].at[:, pl.ds(0, ext)], sems[c % 2]).wait()

        def scalar_at(vec_ref, j):
            grp = vec_ref[pl.ds((j // _LANE) * _LANE, _LANE)]
            return jnp.sum(jnp.where(iota16 == (j % _LANE), grp, 0))

        def unit_body(j, carry):
            u = wid + j * nw

            @pl.when(u < n_units)
            def _():
                f = u // egroups
                g8 = pl.multiple_of((u % egroups) * 8, 8)
                pltpu.sync_copy(sv_hbm.at[f], sv_v)
                pltpu.sync_copy(bnd_hbm.at[f], bnd_v)
                pltpu.sync_copy(reg_hbm.at[f, pl.ds(g8, 8), :], reg_v)

                def process(buf, v0, v1, lo, hi):
                    def group_body(k, c2):
                        off = pl.multiple_of(k * _LANE, 8)
                        pk16 = sv_v[pl.ds(off, _LANE)]
                        sv16 = lax.shift_right_logical(pk16, pos_bits)
                        pos16 = jnp.bitwise_and(pk16, batch - 1)
                        m = jnp.logical_and(sv16 >= v0, sv16 < v1)
                        vloc = jnp.where(m, sv16 - v0, 0)
                        for s in range(8):
                            s16 = jnp.full((_LANE,), s, jnp.int32)
                            vals = plsc.load_gather(buf, [s16, vloc])
                            plsc.store_scatter(slab_v, [s16, pos16], vals, mask=m)
                        return c2

                    lax.fori_loop(lo >> 4, (hi + _LANE - 1) >> 4, group_body, 0)

                fire(f, g8, 0)
                for c in range(n_chunks):
                    if c + 1 < n_chunks:
                        fire(f, g8, c + 1)
                    wait(f, g8, c)
                    v0 = c * _CV
                    v1 = v0 + (_CV if c < n_full else tail_v)
                    process(bufs[c % 2], v0, v1,
                            scalar_at(bnd_v, c), scalar_at(bnd_v, c + 1))
                process(reg_v, v_aligned, vocab,
                        scalar_at(bnd_v, n_chunks), scalar_at(bnd_v, n_chunks + 1))
                row0 = f * embed + g8
                pltpu.sync_copy(slab_v, out_hbm.at[pl.ds(row0, 8), :])

            return carry

        lax.fori_loop(0, units_per_w, unit_body, 0)

    return gather_k


def _mlp_t_body(*refs):
    rt_ref = refs[0]
    out_ref = refs[-1]
    w = refs[1:-1]
    rt = rt_ref[...]
    cdims = (((0,), (0,)), ((), ()))
    n_units = (len(w) - 2) // 4
    for u in range(n_units):
        w1, b1, w2, b2 = w[4 * u : 4 * u + 4]
        h = lax.dot_general(w1[...], rt, cdims, preferred_element_type=jnp.float32)
        h = jnp.maximum(h + b1[...], 0.0)
        h = lax.dot_general(w2[...], h, cdims, preferred_element_type=jnp.float32)
        rt = jnp.maximum(rt + h + b2[...], 0.0)
    wd, bd = w[-2], w[-1]
    logit = lax.dot_general(wd[...], rt, cdims, preferred_element_type=jnp.float32)
    out_ref[...] = jax.nn.sigmoid(logit + bd[...])


def _mlp_t(rt, flat_w, block_c=512):
    stack, batch = rt.shape
    grid = (batch // block_c,)
    full = lambda a: pl.BlockSpec(a.shape, lambda i: (0,) * a.ndim)
    in_specs = [pl.BlockSpec((stack, block_c), lambda i: (0, i))]
    in_specs += [full(a) for a in flat_w]
    return pl.pallas_call(
        _mlp_t_body,
        grid=grid,
        in_specs=in_specs,
        out_specs=pl.BlockSpec((1, block_c), lambda i: (0, i)),
        out_shape=jax.ShapeDtypeStruct((1, batch), jnp.float32),
    )(rt, *flat_w)


def kernel(sparse_inputs, params):
    tables = params["tables"]
    n_fields, vocab, embed = tables.shape
    batch = sparse_inputs.shape[0]
    tbl_t = jnp.transpose(tables, (0, 2, 1))

    idx_t = sparse_inputs.astype(jnp.int32).T
    iota_b = jnp.broadcast_to(
        jnp.arange(batch, dtype=jnp.int32)[None, :], idx_t.shape)
    packed = idx_t * batch + iota_b
    sv = lax.sort(packed, dimension=1)
    v_aligned = (vocab // 128) * 128
    grid = jnp.array(
        list(range(0, v_aligned + 1, _CV))[: v_aligned // _CV + 1]
        + [v_aligned, vocab], dtype=jnp.int32)
    bnd = jnp.sum(idx_t[:, :, None] < grid[None, None, :], axis=1,
                  dtype=jnp.int32)
    bnd = jnp.pad(bnd, ((0, 0), (0, 32 - bnd.shape[1])), mode="edge")
    reg = jnp.pad(tbl_t[:, :, v_aligned:], ((0, 0), (0, 0),
                                            (0, 128 - (vocab - v_aligned))))

    rt = _make_gather(n_fields, embed, vocab, batch)(tbl_t, reg, sv, bnd)

    flat_w = []
    for (w1, b1, w2, b2) in params["res"]:
        flat_w += [w1, b1[:, None], w2, b2[:, None]]
    flat_w += [params["Wd"], params["bd"][:, None]]
    out_t = _mlp_t(rt, tuple(flat_w))
    return out_t.reshape(batch, 1)


# speedup vs baseline: 5.4835x; 1.0649x over previous
import functools

import jax
import jax.numpy as jnp
from jax import lax
from jax.experimental import pallas as pl
from jax.experimental.pallas import tpu as pltpu
from jax.experimental.pallas import tpu_sc as plsc

_LANE = 16
_CV = 5120


@functools.lru_cache(maxsize=None)
def _make_gather(n_fields, f_base, embed, vocab, batch):
    info = plsc.get_sparse_core_info()
    nc, ns = info.num_cores, info.num_subcores
    nw = nc * ns
    egroups = embed // 8
    n_units = n_fields * egroups
    units_per_w = -(-n_units // nw)
    v_aligned = (vocab // 128) * 128
    n_full = v_aligned // _CV
    tail_v = v_aligned - n_full * _CV
    n_chunks = n_full + (1 if tail_v else 0)
    pos_bits = batch.bit_length() - 1
    assert (1 << pos_bits) == batch and vocab * batch < 2**31

    mesh = plsc.VectorSubcoreMesh(core_axis_name="c", subcore_axis_name="s")

    @functools.partial(
        pl.kernel,
        mesh=mesh,
        compiler_params=pltpu.CompilerParams(
            use_tc_tiling_on_sc=True, needs_layout_passes=False),
        out_type=jax.ShapeDtypeStruct((n_fields * embed, batch), jnp.float32),
        scratch_types=[
            pltpu.VMEM((batch,), jnp.int32),
            pltpu.VMEM((32,), jnp.int32),
            pltpu.VMEM((8, _CV), jnp.float32),
            pltpu.VMEM((8, _CV), jnp.float32),
            pltpu.VMEM((8, 128), jnp.float32),
            pltpu.VMEM((8, batch), jnp.float32),
            pltpu.SemaphoreType.DMA,
            pltpu.SemaphoreType.DMA,
        ],
    )
    def gather_k(tbl_hbm, reg_hbm, sv_hbm, bnd_hbm, out_hbm,
                 sv_v, bnd_v, buf0, buf1, reg_v, slab_v, sem0, sem1):
        wid = lax.axis_index("s") * nc + lax.axis_index("c")
        bufs = (buf0, buf1)
        sems = (sem0, sem1)
        iota16 = lax.iota(jnp.int32, _LANE)

        def chunk_src(f, g8, c):
            v0 = c * _CV
            ext = _CV if c < n_full else tail_v
            return tbl_hbm.at[f, pl.ds(g8, 8), pl.ds(v0, ext)], ext

        def fire(f, g8, c):
            src, ext = chunk_src(f, g8, c)
            pltpu.async_copy(src, bufs[c % 2].at[:, pl.ds(0, ext)], sems[c % 2])

        def wait(f, g8, c):
            src, ext = chunk_src(f, g8, c)
            pltpu.make_async_copy(
                src, bufs[c % 2].at[:, pl.ds(0, ext)], sems[c % 2]).wait()

        def scalar_at(vec_ref, j):
            grp = vec_ref[pl.ds((j // _LANE) * _LANE, _LANE)]
            return jnp.sum(jnp.where(iota16 == (j % _LANE), grp, 0))

        def unit_body(j, carry):
            u = wid + j * nw

            @pl.when(u < n_units)
            def _():
                fl = u // egroups
                f = fl + f_base
                g8 = pl.multiple_of((u % egroups) * 8, 8)
                pltpu.sync_copy(sv_hbm.at[fl], sv_v)
                pltpu.sync_copy(bnd_hbm.at[fl], bnd_v)
                pltpu.sync_copy(reg_hbm.at[fl, pl.ds(g8, 8), :], reg_v)

                def process(buf, v0, v1, lo, hi):
                    def group_body(k, c2):
                        off = pl.multiple_of(k * _LANE, 8)
                        pk16 = sv_v[pl.ds(off, _LANE)]
                        sv16 = lax.shift_right_logical(pk16, pos_bits)
                        pos16 = jnp.bitwise_and(pk16, batch - 1)
                        m = jnp.logical_and(sv16 >= v0, sv16 < v1)
                        vloc = jnp.where(m, sv16 - v0, 0)
                        for s in range(8):
                            s16 = jnp.full((_LANE,), s, jnp.int32)
                            vals = plsc.load_gather(buf, [s16, vloc])
                            plsc.store_scatter(slab_v, [s16, pos16], vals, mask=m)
                        return c2

                    lax.fori_loop(lo >> 4, (hi + _LANE - 1) >> 4, group_body, 0)

                fire(f, g8, 0)
                for c in range(n_chunks):
                    if c + 1 < n_chunks:
                        fire(f, g8, c + 1)
                    wait(f, g8, c)
                    v0 = c * _CV
                    v1 = v0 + (_CV if c < n_full else tail_v)
                    process(bufs[c % 2], v0, v1,
                            scalar_at(bnd_v, c), scalar_at(bnd_v, c + 1))
                process(reg_v, v_aligned, vocab,
                        scalar_at(bnd_v, n_chunks), scalar_at(bnd_v, n_chunks + 1))
                row0 = fl * embed + g8
                pltpu.sync_copy(slab_v, out_hbm.at[pl.ds(row0, 8), :])

            return carry

        lax.fori_loop(0, units_per_w, unit_body, 0)

    return gather_k


def _mlp_t_body(*refs):
    rt0_ref, rt1_ref = refs[0], refs[1]
    out_ref = refs[-1]
    w = refs[2:-1]
    rt = jnp.concatenate([rt0_ref[...], rt1_ref[...]], axis=0)
    cdims = (((0,), (0,)), ((), ()))
    n_units = (len(w) - 2) // 4
    for u in range(n_units):
        w1, b1, w2, b2 = w[4 * u : 4 * u + 4]
        h = lax.dot_general(w1[...], rt, cdims, preferred_element_type=jnp.float32)
        h = jnp.maximum(h + b1[...], 0.0)
        h = lax.dot_general(w2[...], h, cdims, preferred_element_type=jnp.float32)
        rt = jnp.maximum(rt + h + b2[...], 0.0)
    wd, bd = w[-2], w[-1]
    logit = lax.dot_general(wd[...], rt, cdims, preferred_element_type=jnp.float32)
    out_ref[...] = jax.nn.sigmoid(logit + bd[...])


def _mlp_t(rt0, rt1, flat_w, block_c=512):
    half, batch = rt0.shape
    grid = (batch // block_c,)
    full = lambda a: pl.BlockSpec(a.shape, lambda i: (0,) * a.ndim)
    in_specs = [pl.BlockSpec((half, block_c), lambda i: (0, i)),
                pl.BlockSpec((half, block_c), lambda i: (0, i))]
    in_specs += [full(a) for a in flat_w]
    return pl.pallas_call(
        _mlp_t_body,
        grid=grid,
        in_specs=in_specs,
        out_specs=pl.BlockSpec((1, block_c), lambda i: (0, i)),
        out_shape=jax.ShapeDtypeStruct((1, batch), jnp.float32),
    )(rt0, rt1, *flat_w)


def kernel(sparse_inputs, params):
    tables = params["tables"]
    n_fields, vocab, embed = tables.shape
    batch = sparse_inputs.shape[0]
    tbl_t = jnp.transpose(tables, (0, 2, 1))

    idx_t = sparse_inputs.astype(jnp.int32).T
    iota_b = jnp.broadcast_to(
        jnp.arange(batch, dtype=jnp.int32)[None, :], idx_t.shape)
    packed = idx_t * batch + iota_b
    v_aligned = (vocab // 128) * 128
    grid = jnp.array(
        list(range(0, v_aligned + 1, _CV))[: v_aligned // _CV + 1]
        + [v_aligned, vocab], dtype=jnp.int32)
    bnd = jnp.sum(idx_t[:, :, None] < grid[None, None, :], axis=1,
                  dtype=jnp.int32)
    bnd = jnp.pad(bnd, ((0, 0), (0, 32 - bnd.shape[1])), mode="edge")
    reg = jnp.pad(tbl_t[:, :, v_aligned:], ((0, 0), (0, 0),
                                            (0, 128 - (vocab - v_aligned))))

    fh = n_fields // 2
    flat_w = []
    for (w1, b1, w2, b2) in params["res"]:
        flat_w += [w1, b1[:, None], w2, b2[:, None]]
    flat_w += [params["Wd"], params["bd"][:, None]]

    halves = []
    for f0, f1 in ((0, fh), (fh, n_fields)):
        sv_h = lax.sort(packed[f0:f1], dimension=1)
        halves.append(_make_gather(f1 - f0, f0, embed, vocab, batch)(
            tbl_t, reg[f0:f1], sv_h, bnd[f0:f1]))
    out_t = _mlp_t(halves[0], halves[1], tuple(flat_w))
    return out_t.reshape(batch, 1)


# speedup vs baseline: 5.5903x; 1.0195x over previous
import functools

import jax
import jax.numpy as jnp
from jax import lax
from jax.experimental import pallas as pl
from jax.experimental.pallas import tpu as pltpu
from jax.experimental.pallas import tpu_sc as plsc

_LANE = 16
_CV = 5120


@functools.lru_cache(maxsize=None)
def _make_gather(n_fields, f_base, embed, vocab, batch):
    info = plsc.get_sparse_core_info()
    nc, ns = info.num_cores, info.num_subcores
    nw = nc * ns
    egroups = embed // 8
    n_units = n_fields * egroups
    units_per_w = -(-n_units // nw)
    v_aligned = (vocab // 128) * 128
    n_full = v_aligned // _CV
    tail_v = v_aligned - n_full * _CV
    n_chunks = n_full + (1 if tail_v else 0)
    pos_bits = batch.bit_length() - 1
    assert (1 << pos_bits) == batch and vocab * batch < 2**31

    mesh = plsc.VectorSubcoreMesh(core_axis_name="c", subcore_axis_name="s")

    @functools.partial(
        pl.kernel,
        mesh=mesh,
        compiler_params=pltpu.CompilerParams(
            use_tc_tiling_on_sc=True, needs_layout_passes=False),
        out_type=jax.ShapeDtypeStruct((n_fields * embed, batch), jnp.float32),
        scratch_types=[
            pltpu.VMEM((batch,), jnp.int32),
            pltpu.VMEM((32,), jnp.int32),
            pltpu.VMEM((_CV // 128, 8, 128), jnp.float32),
            pltpu.VMEM((_CV // 128, 8, 128), jnp.float32),
            pltpu.VMEM((1, 8, 128), jnp.float32),
            pltpu.VMEM((8, batch), jnp.float32),
            pltpu.SemaphoreType.DMA,
            pltpu.SemaphoreType.DMA,
        ],
    )
    def gather_k(tbl_hbm, reg_hbm, sv_hbm, bnd_hbm, out_hbm,
                 sv_v, bnd_v, buf0, buf1, reg_v, slab_v, sem0, sem1):
        wid = lax.axis_index("s") * nc + lax.axis_index("c")
        bufs = (buf0, buf1)
        sems = (sem0, sem1)
        iota16 = lax.iota(jnp.int32, _LANE)

        def tile_copy(f, g8, c, t):
            v0 = pl.multiple_of(c * _CV + t * 128, 128)
            return pltpu.make_async_copy(
                tbl_hbm.at[f, pl.ds(g8, 8), pl.ds(v0, 128)],
                bufs[c % 2].at[t], sems[c % 2])

        def fire(f, g8, c):
            nt = (_CV if c < n_full else tail_v) // 128
            lax.fori_loop(0, nt, lambda t, a: (tile_copy(f, g8, c, t).start(), a)[1], 0)

        def wait(f, g8, c):
            nt = (_CV if c < n_full else tail_v) // 128
            lax.fori_loop(0, nt, lambda t, a: (tile_copy(f, g8, c, t).wait(), a)[1], 0)

        def scalar_at(vec_ref, j):
            grp = vec_ref[pl.ds((j // _LANE) * _LANE, _LANE)]
            return jnp.sum(jnp.where(iota16 == (j % _LANE), grp, 0))

        def unit_body(j, carry):
            u = wid + j * nw

            @pl.when(u < n_units)
            def _():
                fl = u // egroups
                f = fl + f_base
                g8 = pl.multiple_of((u % egroups) * 8, 8)
                pltpu.sync_copy(sv_hbm.at[fl], sv_v)
                pltpu.sync_copy(bnd_hbm.at[fl], bnd_v)
                pltpu.sync_copy(reg_hbm.at[fl, pl.ds(g8, 8), :], reg_v.at[0])

                def process(buf, v0, v1, lo, hi):
                    def group_body(k, c2):
                        off = pl.multiple_of(k * _LANE, 8)
                        pk16 = sv_v[pl.ds(off, _LANE)]
                        sv16 = lax.shift_right_logical(pk16, pos_bits)
                        pos16 = jnp.bitwise_and(pk16, batch - 1)
                        m = jnp.logical_and(sv16 >= v0, sv16 < v1)
                        vloc = jnp.where(m, sv16 - v0, 0)
                        t16 = lax.shift_right_logical(vloc, 7)
                        l16 = jnp.bitwise_and(vloc, 127)
                        for s in range(8):
                            s16 = jnp.full((_LANE,), s, jnp.int32)
                            vals = plsc.load_gather(buf, [t16, s16, l16])
                            plsc.store_scatter(slab_v, [s16, pos16], vals, mask=m)
                        return c2

                    lax.fori_loop(lo >> 4, (hi + _LANE - 1) >> 4, group_body, 0)

                fire(f, g8, 0)
                for c in range(n_chunks):
                    if c + 1 < n_chunks:
                        fire(f, g8, c + 1)
                    wait(f, g8, c)
                    v0 = c * _CV
                    v1 = v0 + (_CV if c < n_full else tail_v)
                    process(bufs[c % 2], v0, v1,
                            scalar_at(bnd_v, c), scalar_at(bnd_v, c + 1))
                process(reg_v, v_aligned, vocab,
                        scalar_at(bnd_v, n_chunks), scalar_at(bnd_v, n_chunks + 1))
                row0 = fl * embed + g8
                pltpu.sync_copy(slab_v, out_hbm.at[pl.ds(row0, 8), :])

            return carry

        lax.fori_loop(0, units_per_w, unit_body, 0)

    return gather_k


def _mlp_t_body(*refs):
    rt0_ref, rt1_ref = refs[0], refs[1]
    out_ref = refs[-1]
    w = refs[2:-1]
    rt = jnp.concatenate([rt0_ref[...], rt1_ref[...]], axis=0)
    cdims = (((0,), (0,)), ((), ()))
    n_units = (len(w) - 2) // 4
    for u in range(n_units):
        w1, b1, w2, b2 = w[4 * u : 4 * u + 4]
        h = lax.dot_general(w1[...], rt, cdims, preferred_element_type=jnp.float32)
        h = jnp.maximum(h + b1[...], 0.0)
        h = lax.dot_general(w2[...], h, cdims, preferred_element_type=jnp.float32)
        rt = jnp.maximum(rt + h + b2[...], 0.0)
    wd, bd = w[-2], w[-1]
    logit = lax.dot_general(wd[...], rt, cdims, preferred_element_type=jnp.float32)
    out_ref[...] = jax.nn.sigmoid(logit + bd[...])


def _mlp_t(rt0, rt1, flat_w, block_c=512):
    half, batch = rt0.shape
    grid = (batch // block_c,)
    full = lambda a: pl.BlockSpec(a.shape, lambda i: (0,) * a.ndim)
    in_specs = [pl.BlockSpec((half, block_c), lambda i: (0, i)),
                pl.BlockSpec((half, block_c), lambda i: (0, i))]
    in_specs += [full(a) for a in flat_w]
    return pl.pallas_call(
        _mlp_t_body,
        grid=grid,
        in_specs=in_specs,
        out_specs=pl.BlockSpec((1, block_c), lambda i: (0, i)),
        out_shape=jax.ShapeDtypeStruct((1, batch), jnp.float32),
    )(rt0, rt1, *flat_w)


def kernel(sparse_inputs, params):
    tables = params["tables"]
    n_fields, vocab, embed = tables.shape
    batch = sparse_inputs.shape[0]
    tbl_t = jnp.transpose(tables, (0, 2, 1))

    idx_t = sparse_inputs.astype(jnp.int32).T
    iota_b = jnp.broadcast_to(
        jnp.arange(batch, dtype=jnp.int32)[None, :], idx_t.shape)
    packed = idx_t * batch + iota_b
    v_aligned = (vocab // 128) * 128
    grid = jnp.array(
        list(range(0, v_aligned + 1, _CV))[: v_aligned // _CV + 1]
        + [v_aligned, vocab], dtype=jnp.int32)
    bnd = jnp.sum(idx_t[:, :, None] < grid[None, None, :], axis=1,
                  dtype=jnp.int32)
    bnd = jnp.pad(bnd, ((0, 0), (0, 32 - bnd.shape[1])), mode="edge")
    reg = jnp.pad(tbl_t[:, :, v_aligned:], ((0, 0), (0, 0),
                                            (0, 128 - (vocab - v_aligned))))

    fh = n_fields // 2
    flat_w = []
    for (w1, b1, w2, b2) in params["res"]:
        flat_w += [w1, b1[:, None], w2, b2[:, None]]
    flat_w += [params["Wd"], params["bd"][:, None]]

    halves = []
    for f0, f1 in ((0, fh), (fh, n_fields)):
        sv_h = lax.sort(packed[f0:f1], dimension=1)
        halves.append(_make_gather(f1 - f0, f0, embed, vocab, batch)(
            tbl_t, reg[f0:f1], sv_h, bnd[f0:f1]))
    out_t = _mlp_t(halves[0], halves[1], tuple(flat_w))
    return out_t.reshape(batch, 1)
